# baseline (speedup 1.0000x reference)
import functools

import jax
import jax.numpy as jnp
from jax import lax
from jax.experimental import pallas as pl
from jax.experimental.pallas import tpu as pltpu
from jax.experimental.pallas import tpu_sc as plsc

NC, NS = 2, 16
NW = NC * NS
LANES = 16
NBUF = 4
MBUF = 2
TC_CW = 2048


def _pair_table(token_table):
    V, D = token_table.shape
    tT = token_table.T
    n_blocks = (V + TC_CW - 1) // TC_CW
    hcw = TC_CW // 2

    def body(a_ref, eye_ref, out_ref):
        z = lax.dot_general(a_ref[...], eye_ref[...], (((0,), (0,)), ((), ())))
        out_ref[...] = jnp.concatenate([z[0:hcw], z[hcw:TC_CW]], axis=1)

    return pl.pallas_call(
        body,
        grid=(n_blocks,),
        in_specs=[
            pl.BlockSpec((D, TC_CW), lambda i: (0, i)),
            pl.BlockSpec((D, D), lambda i: (0, 0)),
        ],
        out_specs=pl.BlockSpec((hcw, 2 * D), lambda i: (i, 0)),
        out_shape=jax.ShapeDtypeStruct((n_blocks * hcw, 2 * D), jnp.float32),
    )(tT, jnp.eye(D, dtype=jnp.float32))


@functools.lru_cache(maxsize=None)
def _build(B, L, V, D):
    cols_per_w = B // NW
    n_groups = L // NBUF
    assert B % NW == 0 and L % NBUF == 0 and D == 64 and cols_per_w == 128

    mesh = plsc.VectorSubcoreMesh(
        core_axis_name="c", subcore_axis_name="s", num_cores=NC, num_subcores=NS
    )

    @functools.partial(
        pl.kernel,
        out_type=jax.ShapeDtypeStruct((L, D // 8, B // 128, 1024), jnp.float32),
        mesh=mesh,
        compiler_params=pltpu.CompilerParams(
            use_tc_tiling_on_sc=False, needs_layout_passes=False
        ),
        scratch_types=[
            pltpu.VMEM((L, cols_per_w), jnp.int32),
            pltpu.VMEM((D, L), jnp.float32),
            pltpu.VMEM((NBUF, cols_per_w), jnp.int32),
            pltpu.VMEM((NBUF, cols_per_w, 128), jnp.float32),
            pltpu.VMEM((MBUF, D // 8, 1024), jnp.float32),
            pltpu.SemaphoreType.DMA((NBUF,)),
            pltpu.SemaphoreType.DMA((MBUF,)),
        ],
    )
    def emb(xt_hbm, tt_hbm, pt_hbm, out_hbm, idx_v, pt_v, ridx_v, buf_v,
            obuf_v, gsem, osem):
        wid = lax.axis_index("s") * NC + lax.axis_index("c")
        c0 = pl.multiple_of(wid * cols_per_w, 8)

        pltpu.sync_copy(xt_hbm.at[:, pl.ds(c0, cols_per_w)], idx_v)
        pltpu.sync_copy(pt_hbm, pt_v)

        iota = lax.iota(jnp.int32, LANES)
        n_j = cols_per_w // LANES

        def start_gather(l, b):
            for j in range(n_j):
                v16 = idx_v[l, pl.ds(j * LANES, LANES)]
                ridx_v[b, pl.ds(j * LANES, LANES)] = (
                    lax.shift_left(lax.shift_right_logical(v16, 11), 10)
                    + (v16 & 1023)
                )
            pltpu.async_copy(
                tt_hbm.at[ridx_v.at[b]], buf_v.at[b], gsem.at[b]
            )

        def wait_gather(b):
            pltpu.make_async_copy(
                tt_hbm.at[ridx_v.at[b]], buf_v.at[b], gsem.at[b]
            ).wait()

        def wait_out(m):
            pltpu.make_async_copy(
                obuf_v.at[m], out_hbm.at[0, :, 0, :], osem.at[m]
            ).wait()

        for b in range(NBUF):
            start_gather(b, b)

        rowc = tuple(iota + (16 * j) for j in range(n_j))

        def group_body(g, carry):
            for b in range(NBUF):
                l = g * NBUF + b
                m = b % MBUF
                wait_gather(b)

                @pl.when(l >= MBUF)
                def _():
                    wait_out(m)

                bufb = buf_v.at[b]
                obufm = obuf_v.at[m]
                bl = jnp.full((LANES,), l, jnp.int32)
                hcol = tuple(
                    (lax.shift_right_logical(
                        idx_v[l, pl.ds(j * LANES, LANES)], 4) & 64)
                    for j in range(n_j)
                )

                def d_body(d, carry2):
                    rows, cols = carry2
                    bd = jnp.full((LANES,), d, jnp.int32)
                    pos = plsc.load_gather(pt_v, [bd, bl])
                    dg = lax.shift_right_logical(d, 3)
                    dbase = (d & 7) * 128
                    for j in range(n_j):
                        val = plsc.load_gather(bufb, [rows[j], cols[j] + bd])
                        obufm[dg, pl.ds(dbase + j * LANES, LANES)] = val + pos
                    return carry2

                lax.fori_loop(0, D, d_body, (rowc, hcol), unroll=8)

                pltpu.async_copy(
                    obufm, out_hbm.at[l, :, wid, :], osem.at[m]
                )

                @pl.when(l + NBUF < L)
                def _():
                    start_gather(l + NBUF, b)

            return carry

        lax.fori_loop(0, n_groups, group_body, 0)

        for m in range(MBUF):
            wait_out(m)

    return emb


def kernel(x, token_table, pos_table):
    B, L = x.shape
    V, D = token_table.shape
    xt = x.T
    tt2 = _pair_table(token_table)
    pt = pos_table.T
    out4 = _build(B, L, V, D)(xt.astype(jnp.int32), tt2, pt)
    o5 = out4.reshape(L, D // 8, B // 128, 8, 128)
    return o5.transpose(2, 4, 0, 1, 3).reshape(B, L, D)


# speedup vs baseline: 1.5533x; 1.5533x over previous
import functools

import jax
import jax.numpy as jnp
from jax import lax
from jax.experimental import pallas as pl
from jax.experimental.pallas import tpu as pltpu
from jax.experimental.pallas import tpu_sc as plsc

NC, NS = 2, 16
NW = NC * NS
LANES = 16
NBUF = 4
MBUF = 2
TC_CW = 2048


def _pair_table(token_table):
    V, D = token_table.shape
    tT = token_table.T
    n_blocks = (V + TC_CW - 1) // TC_CW
    hcw = TC_CW // 2

    def body(a_ref, eye_ref, out_ref):
        z = lax.dot_general(a_ref[...], eye_ref[...], (((0,), (0,)), ((), ())))
        out_ref[...] = jnp.concatenate([z[0:hcw], z[hcw:TC_CW]], axis=1)

    return pl.pallas_call(
        body,
        grid=(n_blocks,),
        in_specs=[
            pl.BlockSpec((D, TC_CW), lambda i: (0, i)),
            pl.BlockSpec((D, D), lambda i: (0, 0)),
        ],
        out_specs=pl.BlockSpec((hcw, 2 * D), lambda i: (i, 0)),
        out_shape=jax.ShapeDtypeStruct((n_blocks * hcw, 2 * D), jnp.float32),
    )(tT, jnp.eye(D, dtype=jnp.float32))


@functools.lru_cache(maxsize=None)
def _build(B, L, V, D):
    cols_per_w = B // NW
    n_groups = L // NBUF
    assert B % NW == 0 and L % NBUF == 0 and D == 64 and cols_per_w == 128

    mesh = plsc.VectorSubcoreMesh(
        core_axis_name="c", subcore_axis_name="s", num_cores=NC, num_subcores=NS
    )

    @functools.partial(
        pl.kernel,
        out_type=jax.ShapeDtypeStruct((L, D // 8, B // 128, 1024), jnp.float32),
        mesh=mesh,
        compiler_params=pltpu.CompilerParams(
            use_tc_tiling_on_sc=False, needs_layout_passes=False
        ),
        scratch_types=[
            pltpu.VMEM((L, cols_per_w), jnp.int32),
            pltpu.VMEM((D, L), jnp.float32),
            pltpu.VMEM((NBUF, cols_per_w), jnp.int32),
            pltpu.VMEM((NBUF, cols_per_w, 128), jnp.float32),
            pltpu.VMEM((MBUF, D // 8, 1024), jnp.float32),
            pltpu.SemaphoreType.DMA((NBUF,)),
            pltpu.SemaphoreType.DMA((MBUF,)),
        ],
    )
    def emb(xt_hbm, tt_hbm, pt_hbm, out_hbm, idx_v, pt_v, ridx_v, buf_v,
            obuf_v, gsem, osem):
        wid = lax.axis_index("s") * NC + lax.axis_index("c")
        c0 = pl.multiple_of(wid * cols_per_w, 8)

        pltpu.sync_copy(xt_hbm.at[:, pl.ds(c0, cols_per_w)], idx_v)
        pltpu.sync_copy(pt_hbm, pt_v)

        iota = lax.iota(jnp.int32, LANES)
        n_j = cols_per_w // LANES

        def start_gather(l, b):
            for j in range(n_j):
                v16 = idx_v[l, pl.ds(j * LANES, LANES)]
                ridx_v[b, pl.ds(j * LANES, LANES)] = (
                    lax.shift_left(lax.shift_right_logical(v16, 11), 10)
                    + (v16 & 1023)
                )
            pltpu.async_copy(
                tt_hbm.at[ridx_v.at[b]], buf_v.at[b], gsem.at[b]
            )

        def wait_gather(b):
            pltpu.make_async_copy(
                tt_hbm.at[ridx_v.at[b]], buf_v.at[b], gsem.at[b]
            ).wait()

        def wait_out(m):
            pltpu.make_async_copy(
                obuf_v.at[m], out_hbm.at[0, :, 0, :], osem.at[m]
            ).wait()

        for b in range(NBUF):
            start_gather(b, b)

        rowc = tuple(iota + (16 * j) for j in range(n_j))

        def group_body(g, carry):
            for b in range(NBUF):
                l = g * NBUF + b
                m = b % MBUF
                wait_gather(b)

                @pl.when(l >= MBUF)
                def _():
                    wait_out(m)

                bufb = buf_v.at[b]
                obufm = obuf_v.at[m]
                bl = jnp.full((LANES,), l, jnp.int32)
                hcol = tuple(
                    (lax.shift_right_logical(
                        idx_v[l, pl.ds(j * LANES, LANES)], 4) & 64)
                    for j in range(n_j)
                )

                @plsc.parallel_loop(0, D, unroll=8, carry=(rowc, hcol))
                def d_body(d, carry2):
                    rows, cols = carry2
                    bd = jnp.full((LANES,), d, jnp.int32)
                    pos = plsc.load_gather(pt_v, [bd, bl])
                    dg = lax.shift_right_logical(d, 3)
                    dbase = (d & 7) * 128
                    for j in range(n_j):
                        val = plsc.load_gather(bufb, [rows[j], cols[j] + bd])
                        obufm[dg, pl.ds(dbase + j * LANES, LANES)] = val + pos
                    return carry2

                pltpu.async_copy(
                    obufm, out_hbm.at[l, :, wid, :], osem.at[m]
                )

                @pl.when(l + NBUF < L)
                def _():
                    start_gather(l + NBUF, b)

            return carry

        lax.fori_loop(0, n_groups, group_body, 0)

        for m in range(MBUF):
            wait_out(m)

    return emb


def kernel(x, token_table, pos_table):
    B, L = x.shape
    V, D = token_table.shape
    xt = x.T
    tt2 = _pair_table(token_table)
    pt = pos_table.T
    out4 = _build(B, L, V, D)(xt.astype(jnp.int32), tt2, pt)
    o5 = out4.reshape(L, D // 8, B // 128, 8, 128)
    return o5.transpose(2, 4, 0, 1, 3).reshape(B, L, D)


# speedup vs baseline: 1.5817x; 1.0183x over previous
import functools

import jax
import jax.numpy as jnp
from jax import lax
from jax.experimental import pallas as pl
from jax.experimental.pallas import tpu as pltpu
from jax.experimental.pallas import tpu_sc as plsc

NC, NS = 2, 16
NW = NC * NS
LANES = 16
NBUF = 4
MBUF = 2
TC_CW = 2048


def _pair_table(token_table):
    V, D = token_table.shape
    tT = token_table.T
    n_blocks = (V + TC_CW - 1) // TC_CW
    hcw = TC_CW // 2

    def body(a_ref, eye_ref, out_ref):
        z = lax.dot_general(a_ref[...], eye_ref[...], (((0,), (0,)), ((), ())))
        out_ref[...] = jnp.concatenate([z[0:hcw], z[hcw:TC_CW]], axis=1)

    return pl.pallas_call(
        body,
        grid=(n_blocks,),
        in_specs=[
            pl.BlockSpec((D, TC_CW), lambda i: (0, i)),
            pl.BlockSpec((D, D), lambda i: (0, 0)),
        ],
        out_specs=pl.BlockSpec((hcw, 2 * D), lambda i: (i, 0)),
        out_shape=jax.ShapeDtypeStruct((n_blocks * hcw, 2 * D), jnp.float32),
    )(tT, jnp.eye(D, dtype=jnp.float32))


@functools.lru_cache(maxsize=None)
def _build(B, L, V, D):
    cols_per_w = B // NW
    n_groups = L // NBUF
    assert B % NW == 0 and L % NBUF == 0 and D == 64 and cols_per_w == 128

    mesh = plsc.VectorSubcoreMesh(
        core_axis_name="c", subcore_axis_name="s", num_cores=NC, num_subcores=NS
    )

    @functools.partial(
        pl.kernel,
        out_type=jax.ShapeDtypeStruct((L, D // 8, B // 128, 1024), jnp.float32),
        mesh=mesh,
        compiler_params=pltpu.CompilerParams(
            use_tc_tiling_on_sc=False, needs_layout_passes=False
        ),
        scratch_types=[
            pltpu.VMEM((L, cols_per_w), jnp.int32),
            pltpu.VMEM((D, L), jnp.float32),
            pltpu.VMEM((NBUF, 2, cols_per_w), jnp.int32),
            pltpu.VMEM((NBUF, 2, cols_per_w, 32), jnp.float32),
            pltpu.VMEM((MBUF, D // 8, 1024), jnp.float32),
            pltpu.SemaphoreType.DMA((NBUF,)),
            pltpu.SemaphoreType.DMA((MBUF,)),
        ],
    )
    def emb(xt_hbm, tt_hbm, pt_hbm, out_hbm, idx_v, pt_v, ridx_v, buf_v,
            obuf_v, gsem, osem):
        wid = lax.axis_index("s") * NC + lax.axis_index("c")
        c0 = pl.multiple_of(wid * cols_per_w, 8)

        pltpu.sync_copy(xt_hbm.at[:, pl.ds(c0, cols_per_w)], idx_v)
        pltpu.sync_copy(pt_hbm, pt_v)

        iota = lax.iota(jnp.int32, LANES)
        n_j = cols_per_w // LANES

        def start_gather(l, b):
            for j in range(n_j):
                v16 = idx_v[l, pl.ds(j * LANES, LANES)]
                r0 = (
                    lax.shift_left(lax.shift_right_logical(v16, 11), 12)
                    + lax.shift_left(v16 & 1023, 2)
                    + (lax.shift_right_logical(v16, 9) & 2)
                )
                ridx_v[b, 0, pl.ds(j * LANES, LANES)] = r0
                ridx_v[b, 1, pl.ds(j * LANES, LANES)] = r0 + 1
            pltpu.async_copy(
                tt_hbm.at[ridx_v.at[b, 0]], buf_v.at[b, 0], gsem.at[b]
            )
            pltpu.async_copy(
                tt_hbm.at[ridx_v.at[b, 1]], buf_v.at[b, 1], gsem.at[b]
            )

        def wait_gather(b):
            pltpu.make_async_copy(
                tt_hbm.at[ridx_v.at[b, 0]], buf_v.at[b, 0], gsem.at[b]
            ).wait()
            pltpu.make_async_copy(
                tt_hbm.at[ridx_v.at[b, 1]], buf_v.at[b, 1], gsem.at[b]
            ).wait()

        def wait_out(m):
            pltpu.make_async_copy(
                obuf_v.at[m], out_hbm.at[0, :, 0, :], osem.at[m]
            ).wait()

        for b in range(NBUF):
            start_gather(b, b)

        rowc = tuple(iota + (16 * j) for j in range(n_j))

        def group_body(g, carry):
            for b in range(NBUF):
                l = g * NBUF + b
                m = b % MBUF
                wait_gather(b)

                @pl.when(l >= MBUF)
                def _():
                    wait_out(m)

                bufa = buf_v.at[b, 0]
                bufc = buf_v.at[b, 1]
                obufm = obuf_v.at[m]
                bl = jnp.full((LANES,), l, jnp.int32)

                @plsc.parallel_loop(0, D // 2, unroll=8, carry=rowc)
                def d_body(d, rows):
                    bd = jnp.full((LANES,), d, jnp.int32)
                    pos = plsc.load_gather(pt_v, [bd, bl])
                    pos2 = plsc.load_gather(pt_v, [bd + 32, bl])
                    dg = lax.shift_right_logical(d, 3)
                    dbase = (d & 7) * 128
                    for j in range(n_j):
                        val = plsc.load_gather(bufa, [rows[j], bd])
                        obufm[dg, pl.ds(dbase + j * LANES, LANES)] = val + pos
                        val2 = plsc.load_gather(bufc, [rows[j], bd])
                        obufm[dg + 4, pl.ds(dbase + j * LANES, LANES)] = (
                            val2 + pos2
                        )
                    return rows

                pltpu.async_copy(
                    obufm, out_hbm.at[l, :, wid, :], osem.at[m]
                )

                @pl.when(l + NBUF < L)
                def _():
                    start_gather(l + NBUF, b)

            return carry

        lax.fori_loop(0, n_groups, group_body, 0)

        for m in range(MBUF):
            wait_out(m)

    return emb


def kernel(x, token_table, pos_table):
    B, L = x.shape
    V, D = token_table.shape
    xt = x.T
    tt2 = _pair_table(token_table)
    tt4 = tt2.reshape(tt2.shape[0] * 4, 32)
    pt = pos_table.T
    out4 = _build(B, L, V, D)(xt.astype(jnp.int32), tt4, pt)
    o5 = out4.reshape(L, D // 8, B // 128, 8, 128)
    return o5.transpose(2, 4, 0, 1, 3).reshape(B, L, D)


# speedup vs baseline: 1.8377x; 1.1619x over previous
import functools

import jax
import jax.numpy as jnp
from jax import lax
from jax.experimental import pallas as pl
from jax.experimental.pallas import tpu as pltpu
from jax.experimental.pallas import tpu_sc as plsc

NC, NS = 2, 16
NW = NC * NS
LANES = 16
NBUF = 4
MBUF = 2
TC_CW = 8192
TC_SH = 13


def _pair_table(token_table):
    V, D = token_table.shape
    tT = token_table.T
    n_blocks = (V + TC_CW - 1) // TC_CW
    hcw = TC_CW // 2

    def body(a_ref, eye_ref, out_ref):
        z = lax.dot_general(a_ref[...], eye_ref[...], (((0,), (0,)), ((), ())))
        out_ref[...] = jnp.concatenate([z[0:hcw], z[hcw:TC_CW]], axis=1)

    return pl.pallas_call(
        body,
        grid=(n_blocks,),
        in_specs=[
            pl.BlockSpec((D, TC_CW), lambda i: (0, i)),
            pl.BlockSpec((D, D), lambda i: (0, 0)),
        ],
        out_specs=pl.BlockSpec((hcw, 2 * D), lambda i: (i, 0)),
        out_shape=jax.ShapeDtypeStruct((n_blocks * hcw, 2 * D), jnp.float32),
    )(tT, jnp.eye(D, dtype=jnp.float32))


@functools.lru_cache(maxsize=None)
def _build(B, L, V, D):
    cols_per_w = B // NW
    n_groups = L // NBUF
    assert B % NW == 0 and L % NBUF == 0 and D == 64 and cols_per_w == 128

    mesh = plsc.VectorSubcoreMesh(
        core_axis_name="c", subcore_axis_name="s", num_cores=NC, num_subcores=NS
    )

    @functools.partial(
        pl.kernel,
        out_type=jax.ShapeDtypeStruct((L, D // 8, B // 128, 1024), jnp.float32),
        mesh=mesh,
        compiler_params=pltpu.CompilerParams(
            use_tc_tiling_on_sc=False, needs_layout_passes=False
        ),
        scratch_types=[
            pltpu.VMEM((L, cols_per_w), jnp.int32),
            pltpu.VMEM((D, L), jnp.float32),
            pltpu.VMEM((NBUF, 2, cols_per_w), jnp.int32),
            pltpu.VMEM((NBUF, 2, cols_per_w, 32), jnp.float32),
            pltpu.VMEM((MBUF, D // 8, 1024), jnp.float32),
            pltpu.SemaphoreType.DMA((NBUF,)),
            pltpu.SemaphoreType.DMA((MBUF,)),
        ],
    )
    def emb(xt_hbm, tt_hbm, pt_hbm, out_hbm, idx_v, pt_v, ridx_v, buf_v,
            obuf_v, gsem, osem):
        wid = lax.axis_index("s") * NC + lax.axis_index("c")
        c0 = pl.multiple_of(wid * cols_per_w, 8)

        pltpu.sync_copy(xt_hbm.at[:, pl.ds(c0, cols_per_w)], idx_v)
        pltpu.sync_copy(pt_hbm, pt_v)

        iota = lax.iota(jnp.int32, LANES)
        n_j = cols_per_w // LANES

        def start_gather(l, b):
            for j in range(n_j):
                v16 = idx_v[l, pl.ds(j * LANES, LANES)]
                r0 = (
                    lax.shift_left(lax.shift_right_logical(v16, TC_SH),
                                   TC_SH + 1)
                    + lax.shift_left(v16 & (TC_CW // 2 - 1), 2)
                    + (lax.shift_right_logical(v16, TC_SH - 2) & 2)
                )
                ridx_v[b, 0, pl.ds(j * LANES, LANES)] = r0
                ridx_v[b, 1, pl.ds(j * LANES, LANES)] = r0 + 1
            pltpu.async_copy(
                tt_hbm.at[ridx_v.at[b, 0]], buf_v.at[b, 0], gsem.at[b]
            )
            pltpu.async_copy(
                tt_hbm.at[ridx_v.at[b, 1]], buf_v.at[b, 1], gsem.at[b]
            )

        def wait_gather(b):
            pltpu.make_async_copy(
                tt_hbm.at[ridx_v.at[b, 0]], buf_v.at[b, 0], gsem.at[b]
            ).wait()
            pltpu.make_async_copy(
                tt_hbm.at[ridx_v.at[b, 1]], buf_v.at[b, 1], gsem.at[b]
            ).wait()

        def wait_out(m):
            pltpu.make_async_copy(
                obuf_v.at[m], out_hbm.at[0, :, 0, :], osem.at[m]
            ).wait()

        for b in range(NBUF):
            start_gather(b, b)

        rowc = tuple(iota + (16 * j) for j in range(n_j))

        def group_body(g, carry):
            for b in range(NBUF):
                l = g * NBUF + b
                m = b % MBUF
                wait_gather(b)

                @pl.when(l >= MBUF)
                def _():
                    wait_out(m)

                bufa = buf_v.at[b, 0]
                bufc = buf_v.at[b, 1]
                obufm = obuf_v.at[m]
                bl = jnp.full((LANES,), l, jnp.int32)

                @plsc.parallel_loop(0, D // 2, unroll=16, carry=rowc)
                def d_body(d, rows):
                    bd = jnp.full((LANES,), d, jnp.int32)
                    pos = plsc.load_gather(pt_v, [bd, bl])
                    pos2 = plsc.load_gather(pt_v, [bd + 32, bl])
                    dg = lax.shift_right_logical(d, 3)
                    dbase = (d & 7) * 128
                    for j in range(n_j):
                        val = plsc.load_gather(bufa, [rows[j], bd])
                        obufm[dg, pl.ds(dbase + j * LANES, LANES)] = val + pos
                        val2 = plsc.load_gather(bufc, [rows[j], bd])
                        obufm[dg + 4, pl.ds(dbase + j * LANES, LANES)] = (
                            val2 + pos2
                        )
                    return rows

                pltpu.async_copy(
                    obufm, out_hbm.at[l, :, wid, :], osem.at[m]
                )

                @pl.when(l + NBUF < L)
                def _():
                    start_gather(l + NBUF, b)

            return carry

        lax.fori_loop(0, n_groups, group_body, 0)

        for m in range(MBUF):
            wait_out(m)

    return emb


def kernel(x, token_table, pos_table):
    B, L = x.shape
    V, D = token_table.shape
    xt = x.T
    tt2 = _pair_table(token_table)
    tt4 = tt2.reshape(tt2.shape[0] * 4, 32)
    pt = pos_table.T
    out4 = _build(B, L, V, D)(xt.astype(jnp.int32), tt4, pt)
    o5 = out4.reshape(L, D // 8, B // 128, 8, 128)
    return o5.transpose(2, 4, 0, 1, 3).reshape(B, L, D)


# speedup vs baseline: 1.8961x; 1.0317x over previous
import functools

import jax
import jax.numpy as jnp
from jax import lax
from jax.experimental import pallas as pl
from jax.experimental.pallas import tpu as pltpu
from jax.experimental.pallas import tpu_sc as plsc

NC, NS = 2, 16
NW = NC * NS
LANES = 16
NBUF = 4
MBUF = 4
TC_CW = 16384
TC_SH = 14


def _pair_table(token_table):
    V, D = token_table.shape
    tT = token_table.T
    n_blocks = (V + TC_CW - 1) // TC_CW
    hcw = TC_CW // 2

    def body(a_ref, eye_ref, out_ref):
        z = lax.dot_general(a_ref[...], eye_ref[...], (((0,), (0,)), ((), ())))
        out_ref[...] = jnp.concatenate([z[0:hcw], z[hcw:TC_CW]], axis=1)

    return pl.pallas_call(
        body,
        grid=(n_blocks,),
        in_specs=[
            pl.BlockSpec((D, TC_CW), lambda i: (0, i)),
            pl.BlockSpec((D, D), lambda i: (0, 0)),
        ],
        out_specs=pl.BlockSpec((hcw, 2 * D), lambda i: (i, 0)),
        out_shape=jax.ShapeDtypeStruct((n_blocks * hcw, 2 * D), jnp.float32),
        compiler_params=pltpu.CompilerParams(
            vmem_limit_bytes=100 * 1024 * 1024
        ),
    )(tT, jnp.eye(D, dtype=jnp.float32))


@functools.lru_cache(maxsize=None)
def _build(B, L, V, D):
    cols_per_w = B // NW
    n_groups = L // NBUF
    assert B % NW == 0 and L % NBUF == 0 and D == 64 and cols_per_w == 128

    mesh = plsc.VectorSubcoreMesh(
        core_axis_name="c", subcore_axis_name="s", num_cores=NC, num_subcores=NS
    )

    @functools.partial(
        pl.kernel,
        out_type=jax.ShapeDtypeStruct((L, D // 8, B // 128, 1024), jnp.float32),
        mesh=mesh,
        compiler_params=pltpu.CompilerParams(
            use_tc_tiling_on_sc=False, needs_layout_passes=False
        ),
        scratch_types=[
            pltpu.VMEM((L, cols_per_w), jnp.int32),
            pltpu.VMEM((D, L), jnp.float32),
            pltpu.VMEM((NBUF, 2, cols_per_w), jnp.int32),
            pltpu.VMEM((NBUF, 2, cols_per_w, 32), jnp.float32),
            pltpu.VMEM((MBUF, D // 8, 1024), jnp.float32),
            pltpu.SemaphoreType.DMA((NBUF,)),
            pltpu.SemaphoreType.DMA((MBUF,)),
        ],
    )
    def emb(xt_hbm, tt_hbm, pt_hbm, out_hbm, idx_v, pt_v, ridx_v, buf_v,
            obuf_v, gsem, osem):
        wid = lax.axis_index("s") * NC + lax.axis_index("c")
        c0 = pl.multiple_of(wid * cols_per_w, 8)

        pltpu.sync_copy(xt_hbm.at[:, pl.ds(c0, cols_per_w)], idx_v)
        pltpu.sync_copy(pt_hbm, pt_v)

        iota = lax.iota(jnp.int32, LANES)
        n_j = cols_per_w // LANES

        def start_gather(l, b):
            for j in range(n_j):
                v16 = idx_v[l, pl.ds(j * LANES, LANES)]
                r0 = (
                    lax.shift_left(lax.shift_right_logical(v16, TC_SH),
                                   TC_SH + 1)
                    + lax.shift_left(v16 & (TC_CW // 2 - 1), 2)
                    + (lax.shift_right_logical(v16, TC_SH - 2) & 2)
                )
                ridx_v[b, 0, pl.ds(j * LANES, LANES)] = r0
                ridx_v[b, 1, pl.ds(j * LANES, LANES)] = r0 + 1
            pltpu.async_copy(
                tt_hbm.at[ridx_v.at[b, 0]], buf_v.at[b, 0], gsem.at[b]
            )
            pltpu.async_copy(
                tt_hbm.at[ridx_v.at[b, 1]], buf_v.at[b, 1], gsem.at[b]
            )

        def wait_gather(b):
            pltpu.make_async_copy(
                tt_hbm.at[ridx_v.at[b, 0]], buf_v.at[b, 0], gsem.at[b]
            ).wait()
            pltpu.make_async_copy(
                tt_hbm.at[ridx_v.at[b, 1]], buf_v.at[b, 1], gsem.at[b]
            ).wait()

        def wait_out(m):
            pltpu.make_async_copy(
                obuf_v.at[m], out_hbm.at[0, :, 0, :], osem.at[m]
            ).wait()

        for b in range(NBUF):
            start_gather(b, b)

        rowc = tuple(iota + (16 * j) for j in range(n_j))

        def group_body(g, carry):
            for b in range(NBUF):
                l = g * NBUF + b
                m = b % MBUF
                wait_gather(b)

                @pl.when(l >= MBUF)
                def _():
                    wait_out(m)

                bufa = buf_v.at[b, 0]
                bufc = buf_v.at[b, 1]
                obufm = obuf_v.at[m]
                bl = jnp.full((LANES,), l, jnp.int32)

                @plsc.parallel_loop(0, D // 2, unroll=16, carry=rowc)
                def d_body(d, rows):
                    bd = jnp.full((LANES,), d, jnp.int32)
                    pos = plsc.load_gather(pt_v, [bd, bl])
                    pos2 = plsc.load_gather(pt_v, [bd + 32, bl])
                    dg = lax.shift_right_logical(d, 3)
                    dbase = (d & 7) * 128
                    for j in range(n_j):
                        val = plsc.load_gather(bufa, [rows[j], bd])
                        obufm[dg, pl.ds(dbase + j * LANES, LANES)] = val + pos
                        val2 = plsc.load_gather(bufc, [rows[j], bd])
                        obufm[dg + 4, pl.ds(dbase + j * LANES, LANES)] = (
                            val2 + pos2
                        )
                    return rows

                pltpu.async_copy(
                    obufm, out_hbm.at[l, :, wid, :], osem.at[m]
                )

                @pl.when(l + NBUF < L)
                def _():
                    start_gather(l + NBUF, b)

            return carry

        lax.fori_loop(0, n_groups, group_body, 0)

        for m in range(MBUF):
            wait_out(m)

    return emb


def kernel(x, token_table, pos_table):
    B, L = x.shape
    V, D = token_table.shape
    xt = x.T
    tt2 = _pair_table(token_table)
    tt4 = tt2.reshape(tt2.shape[0] * 4, 32)
    pt = pos_table.T
    out4 = _build(B, L, V, D)(xt.astype(jnp.int32), tt4, pt)
    o5 = out4.reshape(L, D // 8, B // 128, 8, 128)
    return o5.transpose(2, 4, 0, 1, 3).reshape(B, L, D)


# speedup vs baseline: 1.9315x; 1.0187x over previous
import functools

import jax
import jax.numpy as jnp
from jax import lax
from jax.experimental import pallas as pl
from jax.experimental.pallas import tpu as pltpu
from jax.experimental.pallas import tpu_sc as plsc

NC, NS = 2, 16
NW = NC * NS
LANES = 16
NBUF = 5
MBUF = 5
TC_CW = 16384
TC_SH = 14


def _pair_table(token_table):
    V, D = token_table.shape
    tT = token_table.T
    n_blocks = (V + TC_CW - 1) // TC_CW
    hcw = TC_CW // 2

    def body(a_ref, eye_ref, out_ref):
        z = lax.dot_general(a_ref[...], eye_ref[...], (((0,), (0,)), ((), ())))
        out_ref[...] = jnp.concatenate([z[0:hcw], z[hcw:TC_CW]], axis=1)

    return pl.pallas_call(
        body,
        grid=(n_blocks,),
        in_specs=[
            pl.BlockSpec((D, TC_CW), lambda i: (0, i)),
            pl.BlockSpec((D, D), lambda i: (0, 0)),
        ],
        out_specs=pl.BlockSpec((hcw, 2 * D), lambda i: (i, 0)),
        out_shape=jax.ShapeDtypeStruct((n_blocks * hcw, 2 * D), jnp.float32),
        compiler_params=pltpu.CompilerParams(
            vmem_limit_bytes=100 * 1024 * 1024
        ),
    )(tT, jnp.eye(D, dtype=jnp.float32))


@functools.lru_cache(maxsize=None)
def _build(B, L, V, D):
    cols_per_w = B // NW
    n_groups = L // NBUF
    assert B % NW == 0 and L % NBUF == 0 and D == 64 and cols_per_w == 128

    mesh = plsc.VectorSubcoreMesh(
        core_axis_name="c", subcore_axis_name="s", num_cores=NC, num_subcores=NS
    )

    @functools.partial(
        pl.kernel,
        out_type=jax.ShapeDtypeStruct((L, D // 8, B // 128, 1024), jnp.float32),
        mesh=mesh,
        compiler_params=pltpu.CompilerParams(
            use_tc_tiling_on_sc=False, needs_layout_passes=False
        ),
        scratch_types=[
            pltpu.VMEM((L, cols_per_w), jnp.int32),
            pltpu.VMEM((D, L), jnp.float32),
            pltpu.VMEM((NBUF, 2, cols_per_w), jnp.int32),
            pltpu.VMEM((NBUF, 2, cols_per_w, 32), jnp.float32),
            pltpu.VMEM((MBUF, D // 8, 1024), jnp.float32),
            pltpu.SemaphoreType.DMA((NBUF,)),
            pltpu.SemaphoreType.DMA((MBUF,)),
        ],
    )
    def emb(xt_hbm, tt_hbm, pt_hbm, out_hbm, idx_v, pt_v, ridx_v, buf_v,
            obuf_v, gsem, osem):
        wid = lax.axis_index("s") * NC + lax.axis_index("c")
        c0 = pl.multiple_of(wid * cols_per_w, 8)

        pltpu.sync_copy(xt_hbm.at[:, pl.ds(c0, cols_per_w)], idx_v)
        pltpu.sync_copy(pt_hbm, pt_v)

        iota = lax.iota(jnp.int32, LANES)
        n_j = cols_per_w // LANES

        def start_gather(l, b):
            for j in range(n_j):
                v16 = idx_v[l, pl.ds(j * LANES, LANES)]
                r0 = (
                    lax.shift_left(lax.shift_right_logical(v16, TC_SH),
                                   TC_SH + 1)
                    + lax.shift_left(v16 & (TC_CW // 2 - 1), 2)
                    + (lax.shift_right_logical(v16, TC_SH - 2) & 2)
                )
                ridx_v[b, 0, pl.ds(j * LANES, LANES)] = r0
                ridx_v[b, 1, pl.ds(j * LANES, LANES)] = r0 + 1
            pltpu.async_copy(
                tt_hbm.at[ridx_v.at[b, 0]], buf_v.at[b, 0], gsem.at[b]
            )
            pltpu.async_copy(
                tt_hbm.at[ridx_v.at[b, 1]], buf_v.at[b, 1], gsem.at[b]
            )

        def wait_gather(b):
            pltpu.make_async_copy(
                tt_hbm.at[ridx_v.at[b, 0]], buf_v.at[b, 0], gsem.at[b]
            ).wait()
            pltpu.make_async_copy(
                tt_hbm.at[ridx_v.at[b, 1]], buf_v.at[b, 1], gsem.at[b]
            ).wait()

        def wait_out(m):
            pltpu.make_async_copy(
                obuf_v.at[m], out_hbm.at[0, :, 0, :], osem.at[m]
            ).wait()

        for b in range(NBUF):
            start_gather(b, b)

        rowc = tuple(iota + (16 * j) for j in range(n_j))

        def group_body(g, carry):
            for b in range(NBUF):
                l = g * NBUF + b
                m = b % MBUF
                wait_gather(b)

                @pl.when(l >= MBUF)
                def _():
                    wait_out(m)

                bufa = buf_v.at[b, 0]
                bufc = buf_v.at[b, 1]
                obufm = obuf_v.at[m]
                bl = jnp.full((LANES,), l, jnp.int32)

                @plsc.parallel_loop(0, D // 2, unroll=32, carry=rowc)
                def d_body(d, rows):
                    bd = jnp.full((LANES,), d, jnp.int32)
                    pos = plsc.load_gather(pt_v, [bd, bl])
                    pos2 = plsc.load_gather(pt_v, [bd + 32, bl])
                    dg = lax.shift_right_logical(d, 3)
                    dbase = (d & 7) * 128
                    for j in range(n_j):
                        val = plsc.load_gather(bufa, [rows[j], bd])
                        obufm[dg, pl.ds(dbase + j * LANES, LANES)] = val + pos
                        val2 = plsc.load_gather(bufc, [rows[j], bd])
                        obufm[dg + 4, pl.ds(dbase + j * LANES, LANES)] = (
                            val2 + pos2
                        )
                    return rows

                pltpu.async_copy(
                    obufm, out_hbm.at[l, :, wid, :], osem.at[m]
                )

                @pl.when(l + NBUF < L)
                def _():
                    start_gather(l + NBUF, b)

            return carry

        lax.fori_loop(0, n_groups, group_body, 0)

        for m in range(MBUF):
            wait_out(m)

    return emb


def kernel(x, token_table, pos_table):
    B, L = x.shape
    V, D = token_table.shape
    xt = x.T
    tt2 = _pair_table(token_table)
    tt4 = tt2.reshape(tt2.shape[0] * 4, 32)
    pt = pos_table.T
    out4 = _build(B, L, V, D)(xt.astype(jnp.int32), tt4, pt)
    o5 = out4.reshape(L, D // 8, B // 128, 8, 128)
    return o5.transpose(2, 4, 0, 1, 3).reshape(B, L, D)


# speedup vs baseline: 1.9674x; 1.0186x over previous
import functools

import jax
import jax.numpy as jnp
from jax import lax
from jax.experimental import pallas as pl
from jax.experimental.pallas import tpu as pltpu
from jax.experimental.pallas import tpu_sc as plsc

NC, NS = 2, 16
NW = NC * NS
LANES = 16
NBUF = 5
MBUF = 5
TC_CW = 16384
TC_SH = 14


def _pair_table(token_table):
    V, D = token_table.shape
    tT = token_table.T
    n_blocks = (V + TC_CW - 1) // TC_CW
    hcw = TC_CW // 2

    def body(a_ref, eye_ref, out_ref):
        z = lax.dot_general(a_ref[...], eye_ref[...], (((0,), (0,)), ((), ())))
        out_ref[...] = jnp.concatenate([z[0:hcw], z[hcw:TC_CW]], axis=1)

    return pl.pallas_call(
        body,
        grid=(n_blocks,),
        in_specs=[
            pl.BlockSpec((D, TC_CW), lambda i: (0, i)),
            pl.BlockSpec((D, D), lambda i: (0, 0)),
        ],
        out_specs=pl.BlockSpec((hcw, 2 * D), lambda i: (i, 0)),
        out_shape=jax.ShapeDtypeStruct((n_blocks * hcw, 2 * D), jnp.float32),
        compiler_params=pltpu.CompilerParams(
            vmem_limit_bytes=100 * 1024 * 1024
        ),
    )(tT, jnp.eye(D, dtype=jnp.float32))


@functools.lru_cache(maxsize=None)
def _build(B, L, V, D):
    cols_per_w = B // NW
    n_groups = L // NBUF
    assert B % NW == 0 and L % NBUF == 0 and D == 64 and cols_per_w == 128

    mesh = plsc.VectorSubcoreMesh(
        core_axis_name="c", subcore_axis_name="s", num_cores=NC, num_subcores=NS
    )

    @functools.partial(
        pl.kernel,
        out_type=jax.ShapeDtypeStruct((L, D // 8, B // 128, 1024), jnp.float32),
        mesh=mesh,
        compiler_params=pltpu.CompilerParams(
            use_tc_tiling_on_sc=False, needs_layout_passes=False
        ),
        scratch_types=[
            pltpu.VMEM((L, cols_per_w), jnp.int32),
            pltpu.VMEM((D, L), jnp.float32),
            pltpu.VMEM((NBUF, 2, cols_per_w), jnp.int32),
            pltpu.VMEM((NBUF, 2, cols_per_w, 32), jnp.float32),
            pltpu.VMEM((MBUF, D // 8, 1024), jnp.float32),
            pltpu.SemaphoreType.DMA((NBUF,)),
            pltpu.SemaphoreType.DMA((MBUF,)),
        ],
    )
    def emb(xt_hbm, tt_hbm, pt_hbm, out_hbm, idx_v, pt_v, ridx_v, buf_v,
            obuf_v, gsem, osem):
        wid = lax.axis_index("s") * NC + lax.axis_index("c")
        c0 = pl.multiple_of(wid * cols_per_w, 8)

        pltpu.sync_copy(xt_hbm.at[:, pl.ds(c0, cols_per_w)], idx_v)
        pltpu.sync_copy(pt_hbm, pt_v)

        iota = lax.iota(jnp.int32, LANES)
        n_j = cols_per_w // LANES

        def start_gather(l, b):
            for j in range(n_j):
                v16 = idx_v[l, pl.ds(j * LANES, LANES)]
                r0 = (
                    lax.shift_left(lax.shift_right_logical(v16, TC_SH),
                                   TC_SH + 1)
                    + lax.shift_left(v16 & (TC_CW // 2 - 1), 2)
                    + (lax.shift_right_logical(v16, TC_SH - 2) & 2)
                )
                ridx_v[b, 0, pl.ds(j * LANES, LANES)] = r0
                ridx_v[b, 1, pl.ds(j * LANES, LANES)] = r0 + 1
            pltpu.async_copy(
                tt_hbm.at[ridx_v.at[b, 0]], buf_v.at[b, 0], gsem.at[b]
            )
            pltpu.async_copy(
                tt_hbm.at[ridx_v.at[b, 1]], buf_v.at[b, 1], gsem.at[b]
            )

        def wait_gather(b):
            pltpu.make_async_copy(
                tt_hbm.at[ridx_v.at[b, 0]], buf_v.at[b, 0], gsem.at[b]
            ).wait()
            pltpu.make_async_copy(
                tt_hbm.at[ridx_v.at[b, 1]], buf_v.at[b, 1], gsem.at[b]
            ).wait()

        def wait_out(m):
            pltpu.make_async_copy(
                obuf_v.at[m], out_hbm.at[0, :, 0, :], osem.at[m]
            ).wait()

        for b in range(NBUF):
            start_gather(b, b)

        rowc = tuple(iota + (16 * j) for j in range(n_j))

        def group_body(g, carry):
            for b in range(NBUF):
                l = g * NBUF + b
                m = b % MBUF
                wait_gather(b)

                @pl.when(l >= MBUF)
                def _():
                    wait_out(m)

                bufa = buf_v.at[b, 0]
                bufc = buf_v.at[b, 1]
                obufm = obuf_v.at[m]
                bl = jnp.full((LANES,), l, jnp.int32)

                @plsc.parallel_loop(0, D // 2, unroll=8, carry=rowc)
                def d_body(d, rows):
                    bd = jnp.full((LANES,), d, jnp.int32)
                    pos = plsc.load_gather(pt_v, [bd, bl])
                    pos2 = plsc.load_gather(pt_v, [bd + 32, bl])
                    dg = lax.shift_right_logical(d, 3)
                    dbase = (d & 7) * 128
                    for j in range(n_j):
                        val = plsc.load_gather(bufa, [rows[j], bd])
                        obufm[dg, pl.ds(dbase + j * LANES, LANES)] = val + pos
                        val2 = plsc.load_gather(bufc, [rows[j], bd])
                        obufm[dg + 4, pl.ds(dbase + j * LANES, LANES)] = (
                            val2 + pos2
                        )
                    return rows

                pltpu.async_copy(
                    obufm, out_hbm.at[l, :, wid, :], osem.at[m]
                )

                @pl.when(l + NBUF < L)
                def _():
                    start_gather(l + NBUF, b)

            return carry

        lax.fori_loop(0, n_groups, group_body, 0)

        for m in range(MBUF):
            wait_out(m)

    return emb


def kernel(x, token_table, pos_table):
    B, L = x.shape
    V, D = token_table.shape
    xt = x.T
    tt2 = _pair_table(token_table)
    tt4 = tt2.reshape(tt2.shape[0] * 4, 32)
    pt = pos_table.T
    out4 = _build(B, L, V, D)(xt.astype(jnp.int32), tt4, pt)
    o5 = out4.reshape(L, D // 8, B // 128, 8, 128)
    return o5.transpose(2, 4, 0, 1, 3).reshape(B, L, D)
